# baseline (speedup 1.0000x reference)
import functools

import jax
import jax.numpy as jnp
from jax import lax
from jax.experimental import pallas as pl
from jax.experimental.pallas import tpu as pltpu
from jax.experimental.pallas import tpu_sc as plsc

_NN = 50000
_NC = 10000
_H = 64
_HH = 32
_CH = 128
_F32 = jnp.float32

_RT_SIM = 400
_RT_MEN = 200
_RT_REL = 40
_NNP = _NN + 64
_NCP = _NC + 64


def _init_const_bufs(bufs_2d, bufs_1d):
    for ref, val in bufs_2d:
        n_r, n_c = ref.shape

        def body2(i, _, ref=ref, val=val, n_c=n_c):
            for cc in range(n_c // 16):
                ref[i, pl.ds(cc * 16, 16)] = jnp.full((16,), val, _F32)
            return 0

        lax.fori_loop(0, n_r, body2, 0)
    for ref, val in bufs_1d:
        (n,) = ref.shape

        def body1(i, _, ref=ref, val=val):
            ref[pl.ds(i * 16, 16)] = jnp.full((16,), val, _F32)
            return 0

        lax.fori_loop(0, n // 16, body1, 0)


def _span8(n):
    return ((n // 16) + 7) // 8 * 8


def _zero_spmem(t, accum, zero_v):
    n_rows = accum.shape[0]
    span = _span8(n_rows)
    n_chunk = (span + _CH - 1) // _CH

    def body(k, _):
        base = jnp.minimum(t * span + k * _CH, n_rows - _CH)
        pltpu.sync_copy(zero_v, accum.at[pl.ds(base, _CH)])
        return 0

    lax.fori_loop(0, n_chunk, body, 0)


def _seg_pass(tab, src_hbm, dst_hbm, base_row, src_v, dst_v, accum,
              rows, sem_g, sem_s, n_blocks):
    rpb = src_v.shape[0]
    grp = rpb // 4

    def outer(b, _):
        pltpu.sync_copy(src_hbm.at[pl.ds(base_row + b * rpb, rpb)], src_v)
        pltpu.sync_copy(dst_hbm.at[pl.ds(base_row + b * rpb, rpb)], dst_v)
        for u in range(4):
            pltpu.async_copy(tab.at[src_v.at[u]], rows[u], sem_g[u])

        def body(q, _):
            for u in range(4):
                j = q * 4 + u
                pltpu.make_async_copy(tab.at[src_v.at[j]], rows[u],
                                      sem_g[u]).wait()
                pltpu.async_copy(rows[u], accum.at[dst_v.at[j]], sem_s[u],
                                 add=True)
            for u in range(4):
                j = q * 4 + u
                pltpu.make_async_copy(rows[u], accum.at[dst_v.at[j]],
                                      sem_s[u]).wait()
                pltpu.async_copy(tab.at[src_v.at[j + 4]], rows[u], sem_g[u])
            return 0

        lax.fori_loop(0, grp - 1, body, 0)
        for u in range(4):
            j = rpb - 4 + u
            pltpu.make_async_copy(tab.at[src_v.at[j]], rows[u],
                                  sem_g[u]).wait()
            pltpu.async_copy(rows[u], accum.at[dst_v.at[j]], sem_s[u],
                             add=True)
        for u in range(4):
            j = rpb - 4 + u
            pltpu.make_async_copy(rows[u], accum.at[dst_v.at[j]],
                                  sem_s[u]).wait()
        return 0

    lax.fori_loop(0, n_blocks, outer, 0)


def _cnt_pass(dst_hbm, base_row, dst_v, cnts, ones_v, sem_c, n_blocks):
    rpb = dst_v.shape[0]

    def outer(b, _):
        pltpu.sync_copy(dst_hbm.at[pl.ds(base_row + b * rpb, rpb)], dst_v)

        def body(q, _):
            for u in range(8):
                pltpu.async_copy(ones_v, cnts.at[dst_v.at[q * 8 + u]],
                                 sem_c, add=True)
            for u in range(8):
                pltpu.make_async_copy(ones_v, cnts.at[dst_v.at[q * 8 + u]],
                                      sem_c).wait()
            return 0

        lax.fori_loop(0, rpb // 8, body, 0)
        return 0

    lax.fori_loop(0, n_blocks, outer, 0)


def _copy_out(t, c, accum, out, n_valid):
    span = _span8(n_valid)
    n_chunk = (span + _CH - 1) // _CH

    def body(k, _):
        base = jnp.minimum(t * span + k * _CH, n_valid - _CH)
        pltpu.sync_copy(accum.at[pl.ds(base, _CH)],
                        out.at[c, pl.ds(base, _CH)])
        return 0

    lax.fori_loop(0, n_chunk, body, 0)


def _reduce_copy_out(t, c, accum, bufs, out, n_valid):
    span = _span8(n_valid)
    n_chunk = (span + _CH - 1) // _CH
    n_slab = accum.shape[0] // 4

    def body(k, _):
        base = jnp.minimum(t * span + k * _CH, n_valid - _CH)
        for r in range(4):
            pltpu.sync_copy(accum.at[pl.ds(r * n_slab + base, _CH)], bufs[r])

        def vadd(i, _):
            for h in range(2):
                sl = pl.ds(h * 16, 16)
                bufs[0][i, sl] = (bufs[0][i, sl] + bufs[1][i, sl]
                                  + bufs[2][i, sl] + bufs[3][i, sl])
            return 0

        lax.fori_loop(0, _CH, vadd, 0)
        pltpu.sync_copy(bufs[0], out.at[c, pl.ds(base, _CH)])
        return 0

    lax.fori_loop(0, n_chunk, body, 0)


def _copy_out_flat(t, c, cnts, out, n_valid):
    span = _span8(n_valid)
    n_chunk = (span + _CH - 1) // _CH

    def body(k, _):
        base = jnp.minimum(t * span + k * _CH, n_valid - _CH)
        pltpu.sync_copy(cnts.at[pl.ds(base, _CH)],
                        out.at[pl.ds(c * n_valid + base, _CH)])
        return 0

    lax.fori_loop(0, n_chunk, body, 0)


def _make_cnt_kernel():
    mesh = plsc.VectorSubcoreMesh(core_axis_name="c", subcore_axis_name="s")

    @functools.partial(
        pl.kernel,
        out_type=[jax.ShapeDtypeStruct((2 * _NN,), _F32),
                  jax.ShapeDtypeStruct((2 * _NC,), _F32),
                  jax.ShapeDtypeStruct((2 * _NC,), _F32)],
        mesh=mesh,
        compiler_params=pltpu.CompilerParams(use_tc_tiling_on_sc=False),
        scratch_types=[
            pltpu.VMEM_SHARED((_NNP,), _F32),
            pltpu.VMEM_SHARED((_NCP,), _F32),
            pltpu.VMEM_SHARED((_NCP,), _F32),
            pltpu.VMEM((_RT_REL, _CH), jnp.int32),
            pltpu.VMEM((_CH,), _F32),
            pltpu.VMEM((_CH,), _F32),
            pltpu.SemaphoreType.DMA,
        ],
    )
    def k(sdst_hbm, mdst_hbm, rdst_hbm, cs_out, cm_out, cr_out,
          cs, cm, cr, dst_v, zero1_v, ones_v, sem_c):
        c = lax.axis_index("c")
        s = lax.axis_index("s")
        w = s * 2 + c
        _init_const_bufs([], [(zero1_v, 0.0), (ones_v, 1.0)])
        _zero_spmem(s, cs, zero1_v)
        _zero_spmem(s, cm, zero1_v)
        _zero_spmem(s, cr, zero1_v)
        plsc.subcore_barrier()
        _cnt_pass(sdst_hbm, w * _RT_MEN, dst_v, cs, ones_v, sem_c, 5)
        _cnt_pass(mdst_hbm, w * _RT_MEN, dst_v, cm, ones_v, sem_c, 5)
        _cnt_pass(rdst_hbm, w * _RT_REL, dst_v, cr, ones_v, sem_c, 1)
        plsc.subcore_barrier()
        _copy_out_flat(s, c, cs, cs_out, _NN)
        _copy_out_flat(s, c, cm, cm_out, _NC)
        _copy_out_flat(s, c, cr, cr_out, _NC)

    return k


def _make_sim_kernel():
    mesh = plsc.VectorSubcoreMesh(core_axis_name="c", subcore_axis_name="s")

    @functools.partial(
        pl.kernel,
        out_type=jax.ShapeDtypeStruct((2, _NN, _HH), _F32),
        mesh=mesh,
        compiler_params=pltpu.CompilerParams(use_tc_tiling_on_sc=False),
        scratch_types=[
            pltpu.VMEM_SHARED((_NNP, _HH), _F32),
            pltpu.VMEM((40, _CH), jnp.int32),
            pltpu.VMEM((40, _CH), jnp.int32),
            pltpu.VMEM((_CH, _HH), _F32),
            pltpu.VMEM((_CH, _HH), _F32),
            pltpu.VMEM((_CH, _HH), _F32),
            pltpu.VMEM((_CH, _HH), _F32),
        ] + [pltpu.SemaphoreType.DMA] * 8,
    )
    def k(lo_hbm, hi_hbm, src_hbm, dst_hbm, sum_out,
          accum, src_v, dst_v, r0, r1, r2, r3,
          sg0, sg1, sg2, sg3, ss0, ss1, ss2, ss3):
        rows = [r0, r1, r2, r3]
        sem_g = [sg0, sg1, sg2, sg3]
        sem_s = [ss0, ss1, ss2, ss3]
        c = lax.axis_index("c")
        s = lax.axis_index("s")
        _init_const_bufs([(r0, 0.0)], [])
        _zero_spmem(s, accum, r0)
        plsc.subcore_barrier()

        @pl.when(c == 0)
        def _():
            _seg_pass(lo_hbm, src_hbm, dst_hbm, s * _RT_SIM, src_v, dst_v,
                      accum, rows, sem_g, sem_s, _RT_SIM // 40)

        @pl.when(c == 1)
        def _():
            _seg_pass(hi_hbm, src_hbm, dst_hbm, s * _RT_SIM, src_v, dst_v,
                      accum, rows, sem_g, sem_s, _RT_SIM // 40)

        plsc.subcore_barrier()
        _copy_out(s, c, accum, sum_out, _NN)

    return k


def _make_comp_kernel():
    mesh = plsc.VectorSubcoreMesh(core_axis_name="c", subcore_axis_name="s")

    @functools.partial(
        pl.kernel,
        out_type=[jax.ShapeDtypeStruct((2, _NC, _HH), _F32),
                  jax.ShapeDtypeStruct((2, _NC, _HH), _F32)],
        mesh=mesh,
        compiler_params=pltpu.CompilerParams(use_tc_tiling_on_sc=False),
        scratch_types=[
            pltpu.VMEM_SHARED((4 * _NCP, _HH), _F32),
            pltpu.VMEM_SHARED((_NCP, _HH), _F32),
            pltpu.VMEM((_RT_REL, _CH), jnp.int32),
            pltpu.VMEM((_RT_REL, _CH), jnp.int32),
            pltpu.VMEM((_CH, _HH), _F32),
            pltpu.VMEM((_CH, _HH), _F32),
            pltpu.VMEM((_CH, _HH), _F32),
            pltpu.VMEM((_CH, _HH), _F32),
        ] + [pltpu.SemaphoreType.DMA] * 8,
    )
    def k(xnlo_hbm, xnhi_hbm, xclo_hbm, xchi_hbm,
          msrc_hbm, mdst_hbm, rsrc_hbm, rdst_hbm,
          sum_m, sum_r,
          acc_m, acc_r, src_v, dst_v, r0, r1, r2, r3,
          sg0, sg1, sg2, sg3, ss0, ss1, ss2, ss3):
        rows = [r0, r1, r2, r3]
        sem_g = [sg0, sg1, sg2, sg3]
        sem_s = [ss0, ss1, ss2, ss3]
        c = lax.axis_index("c")
        s = lax.axis_index("s")
        _init_const_bufs([(r0, 0.0)], [])
        _zero_spmem(s, acc_m, r0)
        _zero_spmem(s, acc_r, r0)
        plsc.subcore_barrier()
        mrows = 32 * _RT_MEN // 16
        rrows = 32 * _RT_REL // 16

        @pl.when(c == 0)
        def _():
            _seg_pass(xnlo_hbm, msrc_hbm, mdst_hbm, s * mrows, src_v, dst_v,
                      acc_m, rows, sem_g, sem_s, mrows // _RT_REL)
            _seg_pass(xclo_hbm, rsrc_hbm, rdst_hbm, s * rrows, src_v, dst_v,
                      acc_r, rows, sem_g, sem_s, rrows // _RT_REL)

        @pl.when(c == 1)
        def _():
            _seg_pass(xnhi_hbm, msrc_hbm, mdst_hbm, s * mrows, src_v, dst_v,
                      acc_m, rows, sem_g, sem_s, mrows // _RT_REL)
            _seg_pass(xchi_hbm, rsrc_hbm, rdst_hbm, s * rrows, src_v, dst_v,
                      acc_r, rows, sem_g, sem_s, rrows // _RT_REL)

        plsc.subcore_barrier()
        _reduce_copy_out(s, c, acc_m, rows, sum_m, _NC)
        _copy_out(s, c, acc_r, sum_r, _NC)

    return k


@functools.cache
def _get_sc_kernels():
    return (_make_cnt_kernel(), _make_sim_kernel(), _make_comp_kernel())


def _pad_edges(ei, n_rows_total, n_dst):
    e = ei.shape[1]
    npad = n_rows_total * _CH - e
    pad = jnp.arange(npad, dtype=jnp.int32)
    src = jnp.concatenate([ei[0], pad % 1024]).reshape(n_rows_total, _CH)
    dst = jnp.concatenate([ei[1], n_dst + pad % 64]).reshape(n_rows_total,
                                                             _CH)
    return src, dst


def _ln(x, g, b):
    mu = jnp.mean(x, axis=-1, keepdims=True)
    v = jnp.mean((x - mu) ** 2, axis=-1, keepdims=True)
    return (x - mu) * jax.lax.rsqrt(v + 1e-5) * g + b


def _nproj_body(x_ref, w_ref, b_ref, xn_ref, lo_ref, hi_ref):
    y = jnp.maximum(
        jnp.dot(x_ref[...], w_ref[...], preferred_element_type=_F32)
        + b_ref[...], 0.0)
    xn_ref[...] = y
    lo_ref[...] = y[:, :_HH]
    hi_ref[...] = y[:, _HH:]


def _cproj_body(x_ref, w_ref, b_ref, xc_ref, lo_ref, hi_ref):
    y = jnp.maximum(
        jnp.dot(x_ref[...], w_ref[...], preferred_element_type=_F32)
        + b_ref[...], 0.0)
    xc_ref[...] = y
    lo_ref[...] = y[:, :_HH]
    hi_ref[...] = y[:, _HH:]


def _news_combine_body(lo_ref, hi_ref, c0_ref, c1_ref, xn_ref,
                       wl_ref, bl_ref, wr_ref, g_ref, b_ref,
                       olo_ref, ohi_ref):
    cnt = jnp.maximum(c0_ref[...] + c1_ref[...], 1.0)
    mean = jnp.concatenate([lo_ref[...], hi_ref[...]], axis=1) / cnt
    n1 = (jnp.dot(mean, wl_ref[...], preferred_element_type=_F32)
          + bl_ref[...]
          + jnp.dot(xn_ref[...], wr_ref[...], preferred_element_type=_F32))
    y = _ln(jnp.maximum(n1, 0.0), g_ref[...], b_ref[...])
    olo_ref[...] = y[:, :_HH]
    ohi_ref[...] = y[:, _HH:]


def _comp_means(sm_ref, cm_ref, sr_ref, cr_ref):
    sm = sm_ref[...]
    sr = sr_ref[...]
    mm = (jnp.concatenate([sm[0], sm[1]], axis=1)
          / jnp.maximum(cm_ref[...][0] + cm_ref[...][1], 1.0))
    mr = (jnp.concatenate([sr[0], sr[1]], axis=1)
          / jnp.maximum(cr_ref[...][0] + cr_ref[...][1], 1.0))
    return mm, mr


def _comp_combine_body(sm_ref, cm_ref, sr_ref, cr_ref, xc_ref,
                       wlm_ref, blm_ref, wrm_ref,
                       wlr_ref, blr_ref, wrr_ref,
                       g_ref, b_ref, out_ref, olo_ref, ohi_ref):
    mm, mr = _comp_means(sm_ref, cm_ref, sr_ref, cr_ref)
    xc = xc_ref[...]
    cc = 0.5 * (
        jnp.dot(mm, wlm_ref[...], preferred_element_type=_F32) + blm_ref[...]
        + jnp.dot(xc, wrm_ref[...], preferred_element_type=_F32)
        + jnp.dot(mr, wlr_ref[...], preferred_element_type=_F32)
        + blr_ref[...]
        + jnp.dot(xc, wrr_ref[...], preferred_element_type=_F32))
    y = _ln(jnp.maximum(cc, 0.0), g_ref[...], b_ref[...])
    out_ref[...] = y
    olo_ref[...] = y[:, :_HH]
    ohi_ref[...] = y[:, _HH:]


def _final_body(sm_ref, cm_ref, sr_ref, cr_ref, xc_ref,
                wlm_ref, blm_ref, wrm_ref, wlr_ref, blr_ref, wrr_ref,
                g_ref, b_ref, w1_ref, b1_ref, w2_ref, b2_ref, out_ref):
    mm, mr = _comp_means(sm_ref, cm_ref, sr_ref, cr_ref)
    xc = xc_ref[...]
    cc = 0.5 * (
        jnp.dot(mm, wlm_ref[...], preferred_element_type=_F32) + blm_ref[...]
        + jnp.dot(xc, wrm_ref[...], preferred_element_type=_F32)
        + jnp.dot(mr, wlr_ref[...], preferred_element_type=_F32)
        + blr_ref[...]
        + jnp.dot(xc, wrr_ref[...], preferred_element_type=_F32))
    x2 = _ln(jnp.maximum(cc, 0.0), g_ref[...], b_ref[...])
    h = jnp.maximum(
        jnp.dot(x2, w1_ref[...], preferred_element_type=_F32) + b1_ref[...],
        0.0)
    out_ref[...] = (jnp.dot(h, w2_ref[...], preferred_element_type=_F32)
                    + b2_ref[...])


def kernel(news_x, company_x, edge_sim, edge_men, edge_rel, news_proj_W, news_proj_b, company_proj_W, company_proj_b, c1_sim_Wl, c1_sim_bl, c1_sim_Wr, c1_men_Wl, c1_men_bl, c1_men_Wr, c1_rel_Wl, c1_rel_bl, c1_rel_Wr, c2_sim_Wl, c2_sim_bl, c2_sim_Wr, c2_men_Wl, c2_men_bl, c2_men_Wr, c2_rel_Wl, c2_rel_bl, c2_rel_Wr, ln1n_g, ln1n_b, ln1c_g, ln1c_b, ln2c_g, ln2c_b, cls_W1, cls_b1, cls_W2, cls_b2):
    _cnt_kernel, _sim_kernel, _comp_kernel = _get_sc_kernels()
    sim_src, sim_dst = _pad_edges(edge_sim, 16 * _RT_SIM, _NN)
    men_src, men_dst = _pad_edges(edge_men, 32 * _RT_MEN, _NC)
    rel_src, rel_dst = _pad_edges(edge_rel, 32 * _RT_REL, _NC)
    n_mrows = 32 * _RT_MEN
    men_dst_adj = men_dst + _NCP * (
        (jnp.arange(n_mrows, dtype=jnp.int32) // (n_mrows // 16)) % 4)[:, None]

    sim_cnt, men_cnt, rel_cnt = _cnt_kernel(sim_dst, men_dst, rel_dst)

    news_x, company_x, _, _, _ = lax.optimization_barrier(
        (news_x, company_x, sim_dst, men_dst_adj, rel_dst))

    bm = 5000
    xn, xn_lo, xn_hi = pl.pallas_call(
        _nproj_body,
        grid=(_NN // bm,),
        in_specs=[pl.BlockSpec((bm, 385), lambda i: (i, 0)),
                  pl.BlockSpec((385, _H), lambda i: (0, 0)),
                  pl.BlockSpec((1, _H), lambda i: (0, 0))],
        out_specs=[pl.BlockSpec((bm, _H), lambda i: (i, 0)),
                   pl.BlockSpec((bm, _HH), lambda i: (i, 0)),
                   pl.BlockSpec((bm, _HH), lambda i: (i, 0))],
        out_shape=[jax.ShapeDtypeStruct((_NN, _H), _F32),
                   jax.ShapeDtypeStruct((_NN, _HH), _F32),
                   jax.ShapeDtypeStruct((_NN, _HH), _F32)],
    )(news_x, news_proj_W, news_proj_b.reshape(1, _H))
    xc, xc_lo, xc_hi = pl.pallas_call(
        _cproj_body,
        out_shape=[jax.ShapeDtypeStruct((_NC, _H), _F32),
                   jax.ShapeDtypeStruct((_NC, _HH), _F32),
                   jax.ShapeDtypeStruct((_NC, _HH), _F32)],
    )(company_x, company_proj_W, company_proj_b.reshape(1, _H))

    sim_sum = _sim_kernel(xn_lo, xn_hi, sim_src, sim_dst)
    men_sum, rel_sum = _comp_kernel(xn_lo, xn_hi, xc_lo, xc_hi,
                                    men_src, men_dst_adj, rel_src, rel_dst)

    xn1_lo, xn1_hi = pl.pallas_call(
        _news_combine_body,
        grid=(_NN // bm,),
        in_specs=[pl.BlockSpec((bm, _HH), lambda i: (i, 0)),
                  pl.BlockSpec((bm, _HH), lambda i: (i, 0)),
                  pl.BlockSpec((bm, 1), lambda i: (i, 0)),
                  pl.BlockSpec((bm, 1), lambda i: (i, 0)),
                  pl.BlockSpec((bm, _H), lambda i: (i, 0)),
                  pl.BlockSpec((_H, _H), lambda i: (0, 0)),
                  pl.BlockSpec((1, _H), lambda i: (0, 0)),
                  pl.BlockSpec((_H, _H), lambda i: (0, 0)),
                  pl.BlockSpec((1, _H), lambda i: (0, 0)),
                  pl.BlockSpec((1, _H), lambda i: (0, 0))],
        out_specs=[pl.BlockSpec((bm, _HH), lambda i: (i, 0)),
                   pl.BlockSpec((bm, _HH), lambda i: (i, 0))],
        out_shape=[jax.ShapeDtypeStruct((_NN, _HH), _F32),
                   jax.ShapeDtypeStruct((_NN, _HH), _F32)],
    )(sim_sum[0], sim_sum[1],
      sim_cnt[:_NN].reshape(_NN, 1), sim_cnt[_NN:].reshape(_NN, 1), xn,
      c1_sim_Wl, c1_sim_bl.reshape(1, _H), c1_sim_Wr,
      ln1n_g.reshape(1, _H), ln1n_b.reshape(1, _H))
    bc = 2000
    csp = [pl.BlockSpec((2, bc, _HH), lambda i: (0, i, 0)),
           pl.BlockSpec((2, bc, 1), lambda i: (0, i, 0)),
           pl.BlockSpec((2, bc, _HH), lambda i: (0, i, 0)),
           pl.BlockSpec((2, bc, 1), lambda i: (0, i, 0)),
           pl.BlockSpec((bc, _H), lambda i: (i, 0)),
           pl.BlockSpec((_H, _H), lambda i: (0, 0)),
           pl.BlockSpec((1, _H), lambda i: (0, 0)),
           pl.BlockSpec((_H, _H), lambda i: (0, 0)),
           pl.BlockSpec((_H, _H), lambda i: (0, 0)),
           pl.BlockSpec((1, _H), lambda i: (0, 0)),
           pl.BlockSpec((_H, _H), lambda i: (0, 0)),
           pl.BlockSpec((1, _H), lambda i: (0, 0)),
           pl.BlockSpec((1, _H), lambda i: (0, 0))]
    xc1, xc1_lo, xc1_hi = pl.pallas_call(
        _comp_combine_body,
        grid=(_NC // bc,),
        in_specs=csp,
        out_specs=[pl.BlockSpec((bc, _H), lambda i: (i, 0)),
                   pl.BlockSpec((bc, _HH), lambda i: (i, 0)),
                   pl.BlockSpec((bc, _HH), lambda i: (i, 0))],
        out_shape=[jax.ShapeDtypeStruct((_NC, _H), _F32),
                   jax.ShapeDtypeStruct((_NC, _HH), _F32),
                   jax.ShapeDtypeStruct((_NC, _HH), _F32)],
    )(men_sum, men_cnt.reshape(2, _NC, 1), rel_sum,
      rel_cnt.reshape(2, _NC, 1), xc,
      c1_men_Wl, c1_men_bl.reshape(1, _H), c1_men_Wr,
      c1_rel_Wl, c1_rel_bl.reshape(1, _H), c1_rel_Wr,
      ln1c_g.reshape(1, _H), ln1c_b.reshape(1, _H))

    men_sum2, rel_sum2 = _comp_kernel(xn1_lo, xn1_hi, xc1_lo, xc1_hi,
                                      men_src, men_dst_adj, rel_src, rel_dst)

    fsp = csp + [pl.BlockSpec((_H, 32), lambda i: (0, 0)),
                 pl.BlockSpec((1, 32), lambda i: (0, 0)),
                 pl.BlockSpec((32, 1), lambda i: (0, 0)),
                 pl.BlockSpec((1, 1), lambda i: (0, 0))]
    out = pl.pallas_call(
        _final_body,
        grid=(_NC // bc,),
        in_specs=fsp,
        out_specs=pl.BlockSpec((bc, 1), lambda i: (i, 0)),
        out_shape=jax.ShapeDtypeStruct((_NC, 1), _F32),
    )(men_sum2, men_cnt.reshape(2, _NC, 1), rel_sum2,
      rel_cnt.reshape(2, _NC, 1), xc1,
      c2_men_Wl, c2_men_bl.reshape(1, _H), c2_men_Wr,
      c2_rel_Wl, c2_rel_bl.reshape(1, _H), c2_rel_Wr,
      ln2c_g.reshape(1, _H), ln2c_b.reshape(1, _H),
      cls_W1, cls_b1.reshape(1, 32), cls_W2, cls_b2.reshape(1, 1))
    return out[:, 0]


# speedup vs baseline: 1.1089x; 1.1089x over previous
import functools

import jax
import jax.numpy as jnp
from jax import lax
from jax.experimental import pallas as pl
from jax.experimental.pallas import tpu as pltpu
from jax.experimental.pallas import tpu_sc as plsc

_NN = 50000
_NC = 10000
_H = 64
_HH = 32
_CH = 128
_F32 = jnp.float32

_RT_SIM = 400
_RT_MEN = 200
_RT_REL = 40
_NNP = _NN + 64
_NCP = _NC + 64


def _init_const_bufs(bufs_2d, bufs_1d):
    for ref, val in bufs_2d:
        n_r, n_c = ref.shape

        def body2(i, _, ref=ref, val=val, n_c=n_c):
            for cc in range(n_c // 16):
                ref[i, pl.ds(cc * 16, 16)] = jnp.full((16,), val, _F32)
            return 0

        lax.fori_loop(0, n_r, body2, 0)
    for ref, val in bufs_1d:
        (n,) = ref.shape

        def body1(i, _, ref=ref, val=val):
            ref[pl.ds(i * 16, 16)] = jnp.full((16,), val, _F32)
            return 0

        lax.fori_loop(0, n // 16, body1, 0)


def _span8(n):
    return ((n // 16) + 7) // 8 * 8


def _zero_spmem(t, accum, zero_v):
    n_rows = accum.shape[0]
    span = _span8(n_rows)
    n_chunk = (span + _CH - 1) // _CH

    def body(k, _):
        base = jnp.minimum(t * span + k * _CH, n_rows - _CH)
        pltpu.sync_copy(zero_v, accum.at[pl.ds(base, _CH)])
        return 0

    lax.fori_loop(0, n_chunk, body, 0)


def _seg_pass(tab, src_hbm, dst_hbm, base_row, src_v, dst_v, accum,
              rows, sem_g, sem_s, n_blocks):
    rpb = src_v.shape[0]
    grp = rpb // 4

    def outer(b, _):
        pltpu.sync_copy(src_hbm.at[pl.ds(base_row + b * rpb, rpb)], src_v)
        pltpu.sync_copy(dst_hbm.at[pl.ds(base_row + b * rpb, rpb)], dst_v)
        for u in range(4):
            pltpu.async_copy(tab.at[src_v.at[u]], rows[u], sem_g[u])

        def body(q, _):
            for u in range(4):
                j = q * 4 + u
                pltpu.make_async_copy(tab.at[src_v.at[j]], rows[u],
                                      sem_g[u]).wait()
                pltpu.async_copy(rows[u], accum.at[dst_v.at[j]], sem_s[u],
                                 add=True)
            for u in range(4):
                j = q * 4 + u
                pltpu.make_async_copy(rows[u], accum.at[dst_v.at[j]],
                                      sem_s[u]).wait()
                pltpu.async_copy(tab.at[src_v.at[j + 4]], rows[u], sem_g[u])
            return 0

        lax.fori_loop(0, grp - 1, body, 0)
        for u in range(4):
            j = rpb - 4 + u
            pltpu.make_async_copy(tab.at[src_v.at[j]], rows[u],
                                  sem_g[u]).wait()
            pltpu.async_copy(rows[u], accum.at[dst_v.at[j]], sem_s[u],
                             add=True)
        for u in range(4):
            j = rpb - 4 + u
            pltpu.make_async_copy(rows[u], accum.at[dst_v.at[j]],
                                  sem_s[u]).wait()
        return 0

    lax.fori_loop(0, n_blocks, outer, 0)


def _cnt_pass(dst_hbm, base_row, dst_v, cnts, ones_v, sem_c, n_blocks):
    rpb = dst_v.shape[0]

    def outer(b, _):
        pltpu.sync_copy(dst_hbm.at[pl.ds(base_row + b * rpb, rpb)], dst_v)

        def body(q, _):
            for u in range(8):
                pltpu.async_copy(ones_v, cnts.at[dst_v.at[q * 8 + u]],
                                 sem_c, add=True)
            for u in range(8):
                pltpu.make_async_copy(ones_v, cnts.at[dst_v.at[q * 8 + u]],
                                      sem_c).wait()
            return 0

        lax.fori_loop(0, rpb // 8, body, 0)
        return 0

    lax.fori_loop(0, n_blocks, outer, 0)


def _copy_out(t, c, accum, out, n_valid):
    span = _span8(n_valid)
    n_chunk = (span + _CH - 1) // _CH

    def body(k, _):
        base = jnp.minimum(t * span + k * _CH, n_valid - _CH)
        pltpu.sync_copy(accum.at[pl.ds(base, _CH)],
                        out.at[c, pl.ds(base, _CH)])
        return 0

    lax.fori_loop(0, n_chunk, body, 0)


def _reduce_copy_out(t, c, accum, bufs, out, n_valid):
    span = _span8(n_valid)
    n_chunk = (span + _CH - 1) // _CH
    n_slab = accum.shape[0] // 4

    def body(k, _):
        base = jnp.minimum(t * span + k * _CH, n_valid - _CH)
        for r in range(4):
            pltpu.sync_copy(accum.at[pl.ds(r * n_slab + base, _CH)], bufs[r])

        def vadd(i, _):
            for h in range(2):
                sl = pl.ds(h * 16, 16)
                bufs[0][i, sl] = (bufs[0][i, sl] + bufs[1][i, sl]
                                  + bufs[2][i, sl] + bufs[3][i, sl])
            return 0

        lax.fori_loop(0, _CH, vadd, 0)
        pltpu.sync_copy(bufs[0], out.at[c, pl.ds(base, _CH)])
        return 0

    lax.fori_loop(0, n_chunk, body, 0)


def _copy_out_flat(t, c, cnts, out, n_valid):
    span = _span8(n_valid)
    n_chunk = (span + _CH - 1) // _CH

    def body(k, _):
        base = jnp.minimum(t * span + k * _CH, n_valid - _CH)
        pltpu.sync_copy(cnts.at[pl.ds(base, _CH)],
                        out.at[pl.ds(c * n_valid + base, _CH)])
        return 0

    lax.fori_loop(0, n_chunk, body, 0)


def _make_cnt_kernel():
    mesh = plsc.VectorSubcoreMesh(core_axis_name="c", subcore_axis_name="s")

    @functools.partial(
        pl.kernel,
        out_type=[jax.ShapeDtypeStruct((2 * _NN,), _F32),
                  jax.ShapeDtypeStruct((2 * _NC,), _F32),
                  jax.ShapeDtypeStruct((2 * _NC,), _F32)],
        mesh=mesh,
        compiler_params=pltpu.CompilerParams(use_tc_tiling_on_sc=False),
        scratch_types=[
            pltpu.VMEM_SHARED((_NNP,), _F32),
            pltpu.VMEM_SHARED((_NCP,), _F32),
            pltpu.VMEM_SHARED((_NCP,), _F32),
            pltpu.VMEM((_RT_REL, _CH), jnp.int32),
            pltpu.VMEM((_CH,), _F32),
            pltpu.VMEM((_CH,), _F32),
            pltpu.SemaphoreType.DMA,
        ],
    )
    def k(sdst_hbm, mdst_hbm, rdst_hbm, cs_out, cm_out, cr_out,
          cs, cm, cr, dst_v, zero1_v, ones_v, sem_c):
        c = lax.axis_index("c")
        s = lax.axis_index("s")
        w = s * 2 + c
        _init_const_bufs([], [(zero1_v, 0.0), (ones_v, 1.0)])
        _zero_spmem(s, cs, zero1_v)
        _zero_spmem(s, cm, zero1_v)
        _zero_spmem(s, cr, zero1_v)
        plsc.subcore_barrier()
        _cnt_pass(sdst_hbm, w * _RT_MEN, dst_v, cs, ones_v, sem_c, 5)
        _cnt_pass(mdst_hbm, w * _RT_MEN, dst_v, cm, ones_v, sem_c, 5)
        _cnt_pass(rdst_hbm, w * _RT_REL, dst_v, cr, ones_v, sem_c, 1)
        plsc.subcore_barrier()
        _copy_out_flat(s, c, cs, cs_out, _NN)
        _copy_out_flat(s, c, cm, cm_out, _NC)
        _copy_out_flat(s, c, cr, cr_out, _NC)

    return k


def _make_sim_kernel():
    mesh = plsc.VectorSubcoreMesh(core_axis_name="c", subcore_axis_name="s")

    @functools.partial(
        pl.kernel,
        out_type=jax.ShapeDtypeStruct((2, _NN, _HH), _F32),
        mesh=mesh,
        compiler_params=pltpu.CompilerParams(use_tc_tiling_on_sc=False),
        scratch_types=[
            pltpu.VMEM_SHARED((_NNP, _HH), _F32),
            pltpu.VMEM((40, _CH), jnp.int32),
            pltpu.VMEM((40, _CH), jnp.int32),
            pltpu.VMEM((_CH, _HH), _F32),
            pltpu.VMEM((_CH, _HH), _F32),
            pltpu.VMEM((_CH, _HH), _F32),
            pltpu.VMEM((_CH, _HH), _F32),
        ] + [pltpu.SemaphoreType.DMA] * 8,
    )
    def k(lo_hbm, hi_hbm, src_hbm, dst_hbm, sum_out,
          accum, src_v, dst_v, r0, r1, r2, r3,
          sg0, sg1, sg2, sg3, ss0, ss1, ss2, ss3):
        rows = [r0, r1, r2, r3]
        sem_g = [sg0, sg1, sg2, sg3]
        sem_s = [ss0, ss1, ss2, ss3]
        c = lax.axis_index("c")
        s = lax.axis_index("s")
        _init_const_bufs([(r0, 0.0)], [])
        _zero_spmem(s, accum, r0)
        plsc.subcore_barrier()

        @pl.when(c == 0)
        def _():
            _seg_pass(lo_hbm, src_hbm, dst_hbm, s * _RT_SIM, src_v, dst_v,
                      accum, rows, sem_g, sem_s, _RT_SIM // 40)

        @pl.when(c == 1)
        def _():
            _seg_pass(hi_hbm, src_hbm, dst_hbm, s * _RT_SIM, src_v, dst_v,
                      accum, rows, sem_g, sem_s, _RT_SIM // 40)

        plsc.subcore_barrier()
        _copy_out(s, c, accum, sum_out, _NN)

    return k


def _make_comp_kernel():
    mesh = plsc.VectorSubcoreMesh(core_axis_name="c", subcore_axis_name="s")

    @functools.partial(
        pl.kernel,
        out_type=[jax.ShapeDtypeStruct((2, _NC, _H), _F32),
                  jax.ShapeDtypeStruct((2, _NC, _H), _F32)],
        mesh=mesh,
        compiler_params=pltpu.CompilerParams(use_tc_tiling_on_sc=False),
        scratch_types=[
            pltpu.VMEM_SHARED((_NCP, _H), _F32),
            pltpu.VMEM_SHARED((_NCP, _H), _F32),
            pltpu.VMEM((_RT_REL, _CH), jnp.int32),
            pltpu.VMEM((_RT_REL, _CH), jnp.int32),
            pltpu.VMEM((_CH, _H), _F32),
            pltpu.VMEM((_CH, _H), _F32),
            pltpu.VMEM((_CH, _H), _F32),
            pltpu.VMEM((_CH, _H), _F32),
        ] + [pltpu.SemaphoreType.DMA] * 8,
    )
    def k(xn_hbm, xc_hbm, msrc_hbm, mdst_hbm, rsrc_hbm, rdst_hbm,
          sum_m, sum_r,
          acc_m, acc_r, src_v, dst_v, r0, r1, r2, r3,
          sg0, sg1, sg2, sg3, ss0, ss1, ss2, ss3):
        rows = [r0, r1, r2, r3]
        sem_g = [sg0, sg1, sg2, sg3]
        sem_s = [ss0, ss1, ss2, ss3]
        c = lax.axis_index("c")
        s = lax.axis_index("s")
        w = s * 2 + c
        _init_const_bufs([(r0, 0.0)], [])
        _zero_spmem(s, acc_m, r0)
        _zero_spmem(s, acc_r, r0)
        plsc.subcore_barrier()
        _seg_pass(xn_hbm, msrc_hbm, mdst_hbm, w * _RT_MEN, src_v, dst_v,
                  acc_m, rows, sem_g, sem_s, _RT_MEN // _RT_REL)
        _seg_pass(xc_hbm, rsrc_hbm, rdst_hbm, w * _RT_REL, src_v, dst_v,
                  acc_r, rows, sem_g, sem_s, 1)
        plsc.subcore_barrier()
        _copy_out(s, c, acc_m, sum_m, _NC)
        _copy_out(s, c, acc_r, sum_r, _NC)

    return k


@functools.cache
def _get_sc_kernels():
    return (_make_cnt_kernel(), _make_sim_kernel(), _make_comp_kernel())


def _pad_edges(ei, n_rows_total, n_dst):
    e = ei.shape[1]
    npad = n_rows_total * _CH - e
    pad = jnp.arange(npad, dtype=jnp.int32)
    src = jnp.concatenate([ei[0], pad % 1024]).reshape(n_rows_total, _CH)
    dst = jnp.concatenate([ei[1], n_dst + pad % 64]).reshape(n_rows_total,
                                                             _CH)
    return src, dst


def _ln(x, g, b):
    mu = jnp.mean(x, axis=-1, keepdims=True)
    v = jnp.mean((x - mu) ** 2, axis=-1, keepdims=True)
    return (x - mu) * jax.lax.rsqrt(v + 1e-5) * g + b


def _nproj_body(x_ref, w_ref, b_ref, xn_ref, lo_ref, hi_ref):
    y = jnp.maximum(
        jnp.dot(x_ref[...], w_ref[...], preferred_element_type=_F32)
        + b_ref[...], 0.0)
    xn_ref[...] = y
    lo_ref[...] = y[:, :_HH]
    hi_ref[...] = y[:, _HH:]


def _cproj_body(x_ref, w_ref, b_ref, xc_ref):
    xc_ref[...] = jnp.maximum(
        jnp.dot(x_ref[...], w_ref[...], preferred_element_type=_F32)
        + b_ref[...], 0.0)


def _news_combine_body(lo_ref, hi_ref, c0_ref, c1_ref, xn_ref,
                       wl_ref, bl_ref, wr_ref, g_ref, b_ref, out_ref):
    cnt = jnp.maximum(c0_ref[...] + c1_ref[...], 1.0)
    mean = jnp.concatenate([lo_ref[...], hi_ref[...]], axis=1) / cnt
    n1 = (jnp.dot(mean, wl_ref[...], preferred_element_type=_F32)
          + bl_ref[...]
          + jnp.dot(xn_ref[...], wr_ref[...], preferred_element_type=_F32))
    out_ref[...] = _ln(jnp.maximum(n1, 0.0), g_ref[...], b_ref[...])


def _comp_means(sm_ref, cm_ref, sr_ref, cr_ref):
    sm = sm_ref[...]
    sr = sr_ref[...]
    mm = (sm[0] + sm[1]) / jnp.maximum(cm_ref[...][0] + cm_ref[...][1], 1.0)
    mr = (sr[0] + sr[1]) / jnp.maximum(cr_ref[...][0] + cr_ref[...][1], 1.0)
    return mm, mr


def _comp_combine_body(sm_ref, cm_ref, sr_ref, cr_ref, xc_ref,
                       wlm_ref, blm_ref, wrm_ref,
                       wlr_ref, blr_ref, wrr_ref,
                       g_ref, b_ref, out_ref):
    mm, mr = _comp_means(sm_ref, cm_ref, sr_ref, cr_ref)
    xc = xc_ref[...]
    cc = 0.5 * (
        jnp.dot(mm, wlm_ref[...], preferred_element_type=_F32) + blm_ref[...]
        + jnp.dot(xc, wrm_ref[...], preferred_element_type=_F32)
        + jnp.dot(mr, wlr_ref[...], preferred_element_type=_F32)
        + blr_ref[...]
        + jnp.dot(xc, wrr_ref[...], preferred_element_type=_F32))
    out_ref[...] = _ln(jnp.maximum(cc, 0.0), g_ref[...], b_ref[...])


def _final_body(sm_ref, cm_ref, sr_ref, cr_ref, xc_ref,
                wlm_ref, blm_ref, wrm_ref, wlr_ref, blr_ref, wrr_ref,
                g_ref, b_ref, w1_ref, b1_ref, w2_ref, b2_ref, out_ref):
    mm, mr = _comp_means(sm_ref, cm_ref, sr_ref, cr_ref)
    xc = xc_ref[...]
    cc = 0.5 * (
        jnp.dot(mm, wlm_ref[...], preferred_element_type=_F32) + blm_ref[...]
        + jnp.dot(xc, wrm_ref[...], preferred_element_type=_F32)
        + jnp.dot(mr, wlr_ref[...], preferred_element_type=_F32)
        + blr_ref[...]
        + jnp.dot(xc, wrr_ref[...], preferred_element_type=_F32))
    x2 = _ln(jnp.maximum(cc, 0.0), g_ref[...], b_ref[...])
    h = jnp.maximum(
        jnp.dot(x2, w1_ref[...], preferred_element_type=_F32) + b1_ref[...],
        0.0)
    out_ref[...] = (jnp.dot(h, w2_ref[...], preferred_element_type=_F32)
                    + b2_ref[...])


def kernel(news_x, company_x, edge_sim, edge_men, edge_rel, news_proj_W, news_proj_b, company_proj_W, company_proj_b, c1_sim_Wl, c1_sim_bl, c1_sim_Wr, c1_men_Wl, c1_men_bl, c1_men_Wr, c1_rel_Wl, c1_rel_bl, c1_rel_Wr, c2_sim_Wl, c2_sim_bl, c2_sim_Wr, c2_men_Wl, c2_men_bl, c2_men_Wr, c2_rel_Wl, c2_rel_bl, c2_rel_Wr, ln1n_g, ln1n_b, ln1c_g, ln1c_b, ln2c_g, ln2c_b, cls_W1, cls_b1, cls_W2, cls_b2):
    _cnt_kernel, _sim_kernel, _comp_kernel = _get_sc_kernels()
    sim_src, sim_dst = _pad_edges(edge_sim, 16 * _RT_SIM, _NN)
    men_src, men_dst = _pad_edges(edge_men, 32 * _RT_MEN, _NC)
    rel_src, rel_dst = _pad_edges(edge_rel, 32 * _RT_REL, _NC)
    sim_cnt, men_cnt, rel_cnt = _cnt_kernel(sim_dst, men_dst, rel_dst)

    news_x, company_x, _, _, _ = lax.optimization_barrier(
        (news_x, company_x, sim_dst, men_dst, rel_dst))

    bm = 5000
    xn, xn_lo, xn_hi = pl.pallas_call(
        _nproj_body,
        grid=(_NN // bm,),
        in_specs=[pl.BlockSpec((bm, 385), lambda i: (i, 0)),
                  pl.BlockSpec((385, _H), lambda i: (0, 0)),
                  pl.BlockSpec((1, _H), lambda i: (0, 0))],
        out_specs=[pl.BlockSpec((bm, _H), lambda i: (i, 0)),
                   pl.BlockSpec((bm, _HH), lambda i: (i, 0)),
                   pl.BlockSpec((bm, _HH), lambda i: (i, 0))],
        out_shape=[jax.ShapeDtypeStruct((_NN, _H), _F32),
                   jax.ShapeDtypeStruct((_NN, _HH), _F32),
                   jax.ShapeDtypeStruct((_NN, _HH), _F32)],
    )(news_x, news_proj_W, news_proj_b.reshape(1, _H))
    xc = pl.pallas_call(
        _cproj_body,
        out_shape=jax.ShapeDtypeStruct((_NC, _H), _F32),
    )(company_x, company_proj_W, company_proj_b.reshape(1, _H))

    sim_sum = _sim_kernel(xn_lo, xn_hi, sim_src, sim_dst)
    men_sum, rel_sum = _comp_kernel(xn, xc, men_src, men_dst,
                                    rel_src, rel_dst)

    xn1 = pl.pallas_call(
        _news_combine_body,
        grid=(_NN // bm,),
        in_specs=[pl.BlockSpec((bm, _HH), lambda i: (i, 0)),
                  pl.BlockSpec((bm, _HH), lambda i: (i, 0)),
                  pl.BlockSpec((bm, 1), lambda i: (i, 0)),
                  pl.BlockSpec((bm, 1), lambda i: (i, 0)),
                  pl.BlockSpec((bm, _H), lambda i: (i, 0)),
                  pl.BlockSpec((_H, _H), lambda i: (0, 0)),
                  pl.BlockSpec((1, _H), lambda i: (0, 0)),
                  pl.BlockSpec((_H, _H), lambda i: (0, 0)),
                  pl.BlockSpec((1, _H), lambda i: (0, 0)),
                  pl.BlockSpec((1, _H), lambda i: (0, 0))],
        out_specs=pl.BlockSpec((bm, _H), lambda i: (i, 0)),
        out_shape=jax.ShapeDtypeStruct((_NN, _H), _F32),
    )(sim_sum[0], sim_sum[1],
      sim_cnt[:_NN].reshape(_NN, 1), sim_cnt[_NN:].reshape(_NN, 1), xn,
      c1_sim_Wl, c1_sim_bl.reshape(1, _H), c1_sim_Wr,
      ln1n_g.reshape(1, _H), ln1n_b.reshape(1, _H))
    bc = 2000
    csp = [pl.BlockSpec((2, bc, _H), lambda i: (0, i, 0)),
           pl.BlockSpec((2, bc, 1), lambda i: (0, i, 0)),
           pl.BlockSpec((2, bc, _H), lambda i: (0, i, 0)),
           pl.BlockSpec((2, bc, 1), lambda i: (0, i, 0)),
           pl.BlockSpec((bc, _H), lambda i: (i, 0)),
           pl.BlockSpec((_H, _H), lambda i: (0, 0)),
           pl.BlockSpec((1, _H), lambda i: (0, 0)),
           pl.BlockSpec((_H, _H), lambda i: (0, 0)),
           pl.BlockSpec((_H, _H), lambda i: (0, 0)),
           pl.BlockSpec((1, _H), lambda i: (0, 0)),
           pl.BlockSpec((_H, _H), lambda i: (0, 0)),
           pl.BlockSpec((1, _H), lambda i: (0, 0)),
           pl.BlockSpec((1, _H), lambda i: (0, 0))]
    xc1 = pl.pallas_call(
        _comp_combine_body,
        grid=(_NC // bc,),
        in_specs=csp,
        out_specs=pl.BlockSpec((bc, _H), lambda i: (i, 0)),
        out_shape=jax.ShapeDtypeStruct((_NC, _H), _F32),
    )(men_sum, men_cnt.reshape(2, _NC, 1), rel_sum,
      rel_cnt.reshape(2, _NC, 1), xc,
      c1_men_Wl, c1_men_bl.reshape(1, _H), c1_men_Wr,
      c1_rel_Wl, c1_rel_bl.reshape(1, _H), c1_rel_Wr,
      ln1c_g.reshape(1, _H), ln1c_b.reshape(1, _H))

    men_sum2, rel_sum2 = _comp_kernel(xn1, xc1, men_src, men_dst,
                                      rel_src, rel_dst)

    fsp = csp + [pl.BlockSpec((_H, 32), lambda i: (0, 0)),
                 pl.BlockSpec((1, 32), lambda i: (0, 0)),
                 pl.BlockSpec((32, 1), lambda i: (0, 0)),
                 pl.BlockSpec((1, 1), lambda i: (0, 0))]
    out = pl.pallas_call(
        _final_body,
        grid=(_NC // bc,),
        in_specs=fsp,
        out_specs=pl.BlockSpec((bc, 1), lambda i: (i, 0)),
        out_shape=jax.ShapeDtypeStruct((_NC, 1), _F32),
    )(men_sum2, men_cnt.reshape(2, _NC, 1), rel_sum2,
      rel_cnt.reshape(2, _NC, 1), xc1,
      c2_men_Wl, c2_men_bl.reshape(1, _H), c2_men_Wr,
      c2_rel_Wl, c2_rel_bl.reshape(1, _H), c2_rel_Wr,
      ln2c_g.reshape(1, _H), ln2c_b.reshape(1, _H),
      cls_W1, cls_b1.reshape(1, 32), cls_W2, cls_b2.reshape(1, 1))
    return out[:, 0]


# speedup vs baseline: 1.1157x; 1.0061x over previous
import functools

import jax
import jax.numpy as jnp
from jax import lax
from jax.experimental import pallas as pl
from jax.experimental.pallas import tpu as pltpu
from jax.experimental.pallas import tpu_sc as plsc

_NN = 50000
_NC = 10000
_H = 64
_HH = 32
_CH = 128
_F32 = jnp.float32

_RT_SIM = 400
_RT_MEN = 200
_RT_REL = 40
_NNP = _NN + 64
_NCP = _NC + 64


def _init_const_bufs(bufs_2d, bufs_1d):
    for ref, val in bufs_2d:
        n_r, n_c = ref.shape

        def body2(i, _, ref=ref, val=val, n_c=n_c):
            for cc in range(n_c // 16):
                ref[i, pl.ds(cc * 16, 16)] = jnp.full((16,), val, _F32)
            return 0

        lax.fori_loop(0, n_r, body2, 0)
    for ref, val in bufs_1d:
        (n,) = ref.shape

        def body1(i, _, ref=ref, val=val):
            ref[pl.ds(i * 16, 16)] = jnp.full((16,), val, _F32)
            return 0

        lax.fori_loop(0, n // 16, body1, 0)


def _span8(n):
    return ((n // 16) + 7) // 8 * 8


def _zero_spmem(t, accum, zero_v):
    n_rows = accum.shape[0]
    span = _span8(n_rows)
    n_chunk = (span + _CH - 1) // _CH

    def body(k, _):
        base = jnp.minimum(t * span + k * _CH, n_rows - _CH)
        pltpu.sync_copy(zero_v, accum.at[pl.ds(base, _CH)])
        return 0

    lax.fori_loop(0, n_chunk, body, 0)


def _seg_pass(tab, src_hbm, dst_hbm, base_row, src_v, dst_v, accum,
              rows, sem_g, sem_s, n_blocks):
    rpb = src_v.shape[0]
    grp = rpb // 4

    def outer(b, _):
        pltpu.sync_copy(src_hbm.at[pl.ds(base_row + b * rpb, rpb)], src_v)
        pltpu.sync_copy(dst_hbm.at[pl.ds(base_row + b * rpb, rpb)], dst_v)
        for u in range(4):
            pltpu.async_copy(tab.at[src_v.at[u]], rows[u], sem_g[u])

        def body(q, _):
            for u in range(4):
                j = q * 4 + u
                pltpu.make_async_copy(tab.at[src_v.at[j]], rows[u],
                                      sem_g[u]).wait()
                pltpu.async_copy(rows[u], accum.at[dst_v.at[j]], sem_s[u],
                                 add=True)
            for u in range(4):
                j = q * 4 + u
                pltpu.make_async_copy(rows[u], accum.at[dst_v.at[j]],
                                      sem_s[u]).wait()
                pltpu.async_copy(tab.at[src_v.at[j + 4]], rows[u], sem_g[u])
            return 0

        lax.fori_loop(0, grp - 1, body, 0)
        for u in range(4):
            j = rpb - 4 + u
            pltpu.make_async_copy(tab.at[src_v.at[j]], rows[u],
                                  sem_g[u]).wait()
            pltpu.async_copy(rows[u], accum.at[dst_v.at[j]], sem_s[u],
                             add=True)
        for u in range(4):
            j = rpb - 4 + u
            pltpu.make_async_copy(rows[u], accum.at[dst_v.at[j]],
                                  sem_s[u]).wait()
        return 0

    lax.fori_loop(0, n_blocks, outer, 0)


def _cnt_pass(dst_hbm, base_row, dst_v, cnts, ones_v, sem_c, n_blocks):
    rpb = dst_v.shape[0]

    def outer(b, _):
        pltpu.sync_copy(dst_hbm.at[pl.ds(base_row + b * rpb, rpb)], dst_v)
        for u in range(8):
            pltpu.async_copy(ones_v, cnts.at[dst_v.at[u]], sem_c, add=True)

        def body(j, _):
            pltpu.make_async_copy(ones_v, cnts.at[dst_v.at[j]], sem_c).wait()
            pltpu.async_copy(ones_v, cnts.at[dst_v.at[j + 8]], sem_c,
                             add=True)
            return 0

        lax.fori_loop(0, rpb - 8, body, 0)
        for u in range(8):
            pltpu.make_async_copy(ones_v, cnts.at[dst_v.at[rpb - 8 + u]],
                                  sem_c).wait()
        return 0

    lax.fori_loop(0, n_blocks, outer, 0)


def _copy_out(t, c, accum, out, n_valid):
    span = _span8(n_valid)
    n_chunk = (span + _CH - 1) // _CH

    def body(k, _):
        base = jnp.minimum(t * span + k * _CH, n_valid - _CH)
        pltpu.sync_copy(accum.at[pl.ds(base, _CH)],
                        out.at[c, pl.ds(base, _CH)])
        return 0

    lax.fori_loop(0, n_chunk, body, 0)


def _reduce_copy_out(t, c, accum, bufs, out, n_valid):
    span = _span8(n_valid)
    n_chunk = (span + _CH - 1) // _CH
    n_slab = accum.shape[0] // 4

    def body(k, _):
        base = jnp.minimum(t * span + k * _CH, n_valid - _CH)
        for r in range(4):
            pltpu.sync_copy(accum.at[pl.ds(r * n_slab + base, _CH)], bufs[r])

        def vadd(i, _):
            for h in range(2):
                sl = pl.ds(h * 16, 16)
                bufs[0][i, sl] = (bufs[0][i, sl] + bufs[1][i, sl]
                                  + bufs[2][i, sl] + bufs[3][i, sl])
            return 0

        lax.fori_loop(0, _CH, vadd, 0)
        pltpu.sync_copy(bufs[0], out.at[c, pl.ds(base, _CH)])
        return 0

    lax.fori_loop(0, n_chunk, body, 0)


def _copy_out_flat(t, c, cnts, out, n_valid):
    span = _span8(n_valid)
    n_chunk = (span + _CH - 1) // _CH

    def body(k, _):
        base = jnp.minimum(t * span + k * _CH, n_valid - _CH)
        pltpu.sync_copy(cnts.at[pl.ds(base, _CH)],
                        out.at[pl.ds(c * n_valid + base, _CH)])
        return 0

    lax.fori_loop(0, n_chunk, body, 0)


def _make_cnt_kernel():
    mesh = plsc.VectorSubcoreMesh(core_axis_name="c", subcore_axis_name="s")

    @functools.partial(
        pl.kernel,
        out_type=[jax.ShapeDtypeStruct((2 * _NN,), _F32),
                  jax.ShapeDtypeStruct((2 * _NC,), _F32),
                  jax.ShapeDtypeStruct((2 * _NC,), _F32)],
        mesh=mesh,
        compiler_params=pltpu.CompilerParams(use_tc_tiling_on_sc=False),
        scratch_types=[
            pltpu.VMEM_SHARED((_NNP,), _F32),
            pltpu.VMEM_SHARED((_NCP,), _F32),
            pltpu.VMEM_SHARED((_NCP,), _F32),
            pltpu.VMEM((_RT_REL, _CH), jnp.int32),
            pltpu.VMEM((_CH,), _F32),
            pltpu.VMEM((_CH,), _F32),
            pltpu.SemaphoreType.DMA,
        ],
    )
    def k(sdst_hbm, mdst_hbm, rdst_hbm, cs_out, cm_out, cr_out,
          cs, cm, cr, dst_v, zero1_v, ones_v, sem_c):
        c = lax.axis_index("c")
        s = lax.axis_index("s")
        w = s * 2 + c
        _init_const_bufs([], [(zero1_v, 0.0), (ones_v, 1.0)])
        _zero_spmem(s, cs, zero1_v)
        _zero_spmem(s, cm, zero1_v)
        _zero_spmem(s, cr, zero1_v)
        plsc.subcore_barrier()
        _cnt_pass(sdst_hbm, w * _RT_MEN, dst_v, cs, ones_v, sem_c, 5)
        _cnt_pass(mdst_hbm, w * _RT_MEN, dst_v, cm, ones_v, sem_c, 5)
        _cnt_pass(rdst_hbm, w * _RT_REL, dst_v, cr, ones_v, sem_c, 1)
        plsc.subcore_barrier()
        _copy_out_flat(s, c, cs, cs_out, _NN)
        _copy_out_flat(s, c, cm, cm_out, _NC)
        _copy_out_flat(s, c, cr, cr_out, _NC)

    return k


def _make_sim_kernel():
    mesh = plsc.VectorSubcoreMesh(core_axis_name="c", subcore_axis_name="s")

    @functools.partial(
        pl.kernel,
        out_type=jax.ShapeDtypeStruct((2, _NN, _HH), _F32),
        mesh=mesh,
        compiler_params=pltpu.CompilerParams(use_tc_tiling_on_sc=False),
        scratch_types=[
            pltpu.VMEM_SHARED((_NNP, _HH), _F32),
            pltpu.VMEM((40, _CH), jnp.int32),
            pltpu.VMEM((40, _CH), jnp.int32),
            pltpu.VMEM((_CH, _HH), _F32),
            pltpu.VMEM((_CH, _HH), _F32),
            pltpu.VMEM((_CH, _HH), _F32),
            pltpu.VMEM((_CH, _HH), _F32),
        ] + [pltpu.SemaphoreType.DMA] * 8,
    )
    def k(lo_hbm, hi_hbm, src_hbm, dst_hbm, sum_out,
          accum, src_v, dst_v, r0, r1, r2, r3,
          sg0, sg1, sg2, sg3, ss0, ss1, ss2, ss3):
        rows = [r0, r1, r2, r3]
        sem_g = [sg0, sg1, sg2, sg3]
        sem_s = [ss0, ss1, ss2, ss3]
        c = lax.axis_index("c")
        s = lax.axis_index("s")
        _init_const_bufs([(r0, 0.0)], [])
        _zero_spmem(s, accum, r0)
        plsc.subcore_barrier()

        @pl.when(c == 0)
        def _():
            _seg_pass(lo_hbm, src_hbm, dst_hbm, s * _RT_SIM, src_v, dst_v,
                      accum, rows, sem_g, sem_s, _RT_SIM // 40)

        @pl.when(c == 1)
        def _():
            _seg_pass(hi_hbm, src_hbm, dst_hbm, s * _RT_SIM, src_v, dst_v,
                      accum, rows, sem_g, sem_s, _RT_SIM // 40)

        plsc.subcore_barrier()
        _copy_out(s, c, accum, sum_out, _NN)

    return k


def _make_comp_kernel():
    mesh = plsc.VectorSubcoreMesh(core_axis_name="c", subcore_axis_name="s")

    @functools.partial(
        pl.kernel,
        out_type=[jax.ShapeDtypeStruct((2, _NC, _H), _F32),
                  jax.ShapeDtypeStruct((2, _NC, _H), _F32)],
        mesh=mesh,
        compiler_params=pltpu.CompilerParams(use_tc_tiling_on_sc=False),
        scratch_types=[
            pltpu.VMEM_SHARED((_NCP, _H), _F32),
            pltpu.VMEM_SHARED((_NCP, _H), _F32),
            pltpu.VMEM((_RT_REL, _CH), jnp.int32),
            pltpu.VMEM((_RT_REL, _CH), jnp.int32),
            pltpu.VMEM((_CH, _H), _F32),
            pltpu.VMEM((_CH, _H), _F32),
            pltpu.VMEM((_CH, _H), _F32),
            pltpu.VMEM((_CH, _H), _F32),
        ] + [pltpu.SemaphoreType.DMA] * 8,
    )
    def k(xn_hbm, xc_hbm, msrc_hbm, mdst_hbm, rsrc_hbm, rdst_hbm,
          sum_m, sum_r,
          acc_m, acc_r, src_v, dst_v, r0, r1, r2, r3,
          sg0, sg1, sg2, sg3, ss0, ss1, ss2, ss3):
        rows = [r0, r1, r2, r3]
        sem_g = [sg0, sg1, sg2, sg3]
        sem_s = [ss0, ss1, ss2, ss3]
        c = lax.axis_index("c")
        s = lax.axis_index("s")
        w = s * 2 + c
        _init_const_bufs([(r0, 0.0)], [])
        _zero_spmem(s, acc_m, r0)
        _zero_spmem(s, acc_r, r0)
        plsc.subcore_barrier()
        _seg_pass(xn_hbm, msrc_hbm, mdst_hbm, w * _RT_MEN, src_v, dst_v,
                  acc_m, rows, sem_g, sem_s, _RT_MEN // _RT_REL)
        _seg_pass(xc_hbm, rsrc_hbm, rdst_hbm, w * _RT_REL, src_v, dst_v,
                  acc_r, rows, sem_g, sem_s, 1)
        plsc.subcore_barrier()
        _copy_out(s, c, acc_m, sum_m, _NC)
        _copy_out(s, c, acc_r, sum_r, _NC)

    return k


@functools.cache
def _get_sc_kernels():
    return (_make_cnt_kernel(), _make_sim_kernel(), _make_comp_kernel())


def _pad_edges(ei, n_rows_total, n_dst):
    e = ei.shape[1]
    npad = n_rows_total * _CH - e
    pad = jnp.arange(npad, dtype=jnp.int32)
    src = jnp.concatenate([ei[0], pad % 1024]).reshape(n_rows_total, _CH)
    dst = jnp.concatenate([ei[1], n_dst + pad % 64]).reshape(n_rows_total,
                                                             _CH)
    return src, dst


def _ln(x, g, b):
    mu = jnp.mean(x, axis=-1, keepdims=True)
    v = jnp.mean((x - mu) ** 2, axis=-1, keepdims=True)
    return (x - mu) * jax.lax.rsqrt(v + 1e-5) * g + b


def _nproj_body(x_ref, w_ref, b_ref, xn_ref, lo_ref, hi_ref):
    y = jnp.maximum(
        jnp.dot(x_ref[...], w_ref[...], preferred_element_type=_F32)
        + b_ref[...], 0.0)
    xn_ref[...] = y
    lo_ref[...] = y[:, :_HH]
    hi_ref[...] = y[:, _HH:]


def _cproj_body(x_ref, w_ref, b_ref, xc_ref):
    xc_ref[...] = jnp.maximum(
        jnp.dot(x_ref[...], w_ref[...], preferred_element_type=_F32)
        + b_ref[...], 0.0)


def _news_combine_body(lo_ref, hi_ref, c0_ref, c1_ref, xn_ref,
                       wl_ref, bl_ref, wr_ref, g_ref, b_ref, out_ref):
    cnt = jnp.maximum(c0_ref[...] + c1_ref[...], 1.0)
    mean = jnp.concatenate([lo_ref[...], hi_ref[...]], axis=1) / cnt
    n1 = (jnp.dot(mean, wl_ref[...], preferred_element_type=_F32)
          + bl_ref[...]
          + jnp.dot(xn_ref[...], wr_ref[...], preferred_element_type=_F32))
    out_ref[...] = _ln(jnp.maximum(n1, 0.0), g_ref[...], b_ref[...])


def _comp_means(sm_ref, cm_ref, sr_ref, cr_ref):
    sm = sm_ref[...]
    sr = sr_ref[...]
    mm = (sm[0] + sm[1]) / jnp.maximum(cm_ref[...][0] + cm_ref[...][1], 1.0)
    mr = (sr[0] + sr[1]) / jnp.maximum(cr_ref[...][0] + cr_ref[...][1], 1.0)
    return mm, mr


def _comp_combine_body(sm_ref, cm_ref, sr_ref, cr_ref, xc_ref,
                       wlm_ref, blm_ref, wrm_ref,
                       wlr_ref, blr_ref, wrr_ref,
                       g_ref, b_ref, out_ref):
    mm, mr = _comp_means(sm_ref, cm_ref, sr_ref, cr_ref)
    xc = xc_ref[...]
    cc = 0.5 * (
        jnp.dot(mm, wlm_ref[...], preferred_element_type=_F32) + blm_ref[...]
        + jnp.dot(xc, wrm_ref[...], preferred_element_type=_F32)
        + jnp.dot(mr, wlr_ref[...], preferred_element_type=_F32)
        + blr_ref[...]
        + jnp.dot(xc, wrr_ref[...], preferred_element_type=_F32))
    out_ref[...] = _ln(jnp.maximum(cc, 0.0), g_ref[...], b_ref[...])


def _final_body(sm_ref, cm_ref, sr_ref, cr_ref, xc_ref,
                wlm_ref, blm_ref, wrm_ref, wlr_ref, blr_ref, wrr_ref,
                g_ref, b_ref, w1_ref, b1_ref, w2_ref, b2_ref, out_ref):
    mm, mr = _comp_means(sm_ref, cm_ref, sr_ref, cr_ref)
    xc = xc_ref[...]
    cc = 0.5 * (
        jnp.dot(mm, wlm_ref[...], preferred_element_type=_F32) + blm_ref[...]
        + jnp.dot(xc, wrm_ref[...], preferred_element_type=_F32)
        + jnp.dot(mr, wlr_ref[...], preferred_element_type=_F32)
        + blr_ref[...]
        + jnp.dot(xc, wrr_ref[...], preferred_element_type=_F32))
    x2 = _ln(jnp.maximum(cc, 0.0), g_ref[...], b_ref[...])
    h = jnp.maximum(
        jnp.dot(x2, w1_ref[...], preferred_element_type=_F32) + b1_ref[...],
        0.0)
    out_ref[...] = (jnp.dot(h, w2_ref[...], preferred_element_type=_F32)
                    + b2_ref[...])


def kernel(news_x, company_x, edge_sim, edge_men, edge_rel, news_proj_W, news_proj_b, company_proj_W, company_proj_b, c1_sim_Wl, c1_sim_bl, c1_sim_Wr, c1_men_Wl, c1_men_bl, c1_men_Wr, c1_rel_Wl, c1_rel_bl, c1_rel_Wr, c2_sim_Wl, c2_sim_bl, c2_sim_Wr, c2_men_Wl, c2_men_bl, c2_men_Wr, c2_rel_Wl, c2_rel_bl, c2_rel_Wr, ln1n_g, ln1n_b, ln1c_g, ln1c_b, ln2c_g, ln2c_b, cls_W1, cls_b1, cls_W2, cls_b2):
    _cnt_kernel, _sim_kernel, _comp_kernel = _get_sc_kernels()
    sim_src, sim_dst = _pad_edges(edge_sim, 16 * _RT_SIM, _NN)
    men_src, men_dst = _pad_edges(edge_men, 32 * _RT_MEN, _NC)
    rel_src, rel_dst = _pad_edges(edge_rel, 32 * _RT_REL, _NC)
    sim_cnt, men_cnt, rel_cnt = _cnt_kernel(sim_dst, men_dst, rel_dst)

    news_x, company_x, _, _, _ = lax.optimization_barrier(
        (news_x, company_x, sim_dst, men_dst, rel_dst))

    bm = 5000
    xn, xn_lo, xn_hi = pl.pallas_call(
        _nproj_body,
        grid=(_NN // bm,),
        in_specs=[pl.BlockSpec((bm, 385), lambda i: (i, 0)),
                  pl.BlockSpec((385, _H), lambda i: (0, 0)),
                  pl.BlockSpec((1, _H), lambda i: (0, 0))],
        out_specs=[pl.BlockSpec((bm, _H), lambda i: (i, 0)),
                   pl.BlockSpec((bm, _HH), lambda i: (i, 0)),
                   pl.BlockSpec((bm, _HH), lambda i: (i, 0))],
        out_shape=[jax.ShapeDtypeStruct((_NN, _H), _F32),
                   jax.ShapeDtypeStruct((_NN, _HH), _F32),
                   jax.ShapeDtypeStruct((_NN, _HH), _F32)],
    )(news_x, news_proj_W, news_proj_b.reshape(1, _H))
    xc = pl.pallas_call(
        _cproj_body,
        out_shape=jax.ShapeDtypeStruct((_NC, _H), _F32),
    )(company_x, company_proj_W, company_proj_b.reshape(1, _H))

    sim_sum = _sim_kernel(xn_lo, xn_hi, sim_src, sim_dst)
    men_sum, rel_sum = _comp_kernel(xn, xc, men_src, men_dst,
                                    rel_src, rel_dst)

    xn1 = pl.pallas_call(
        _news_combine_body,
        grid=(_NN // bm,),
        in_specs=[pl.BlockSpec((bm, _HH), lambda i: (i, 0)),
                  pl.BlockSpec((bm, _HH), lambda i: (i, 0)),
                  pl.BlockSpec((bm, 1), lambda i: (i, 0)),
                  pl.BlockSpec((bm, 1), lambda i: (i, 0)),
                  pl.BlockSpec((bm, _H), lambda i: (i, 0)),
                  pl.BlockSpec((_H, _H), lambda i: (0, 0)),
                  pl.BlockSpec((1, _H), lambda i: (0, 0)),
                  pl.BlockSpec((_H, _H), lambda i: (0, 0)),
                  pl.BlockSpec((1, _H), lambda i: (0, 0)),
                  pl.BlockSpec((1, _H), lambda i: (0, 0))],
        out_specs=pl.BlockSpec((bm, _H), lambda i: (i, 0)),
        out_shape=jax.ShapeDtypeStruct((_NN, _H), _F32),
    )(sim_sum[0], sim_sum[1],
      sim_cnt[:_NN].reshape(_NN, 1), sim_cnt[_NN:].reshape(_NN, 1), xn,
      c1_sim_Wl, c1_sim_bl.reshape(1, _H), c1_sim_Wr,
      ln1n_g.reshape(1, _H), ln1n_b.reshape(1, _H))
    bc = 2000
    csp = [pl.BlockSpec((2, bc, _H), lambda i: (0, i, 0)),
           pl.BlockSpec((2, bc, 1), lambda i: (0, i, 0)),
           pl.BlockSpec((2, bc, _H), lambda i: (0, i, 0)),
           pl.BlockSpec((2, bc, 1), lambda i: (0, i, 0)),
           pl.BlockSpec((bc, _H), lambda i: (i, 0)),
           pl.BlockSpec((_H, _H), lambda i: (0, 0)),
           pl.BlockSpec((1, _H), lambda i: (0, 0)),
           pl.BlockSpec((_H, _H), lambda i: (0, 0)),
           pl.BlockSpec((_H, _H), lambda i: (0, 0)),
           pl.BlockSpec((1, _H), lambda i: (0, 0)),
           pl.BlockSpec((_H, _H), lambda i: (0, 0)),
           pl.BlockSpec((1, _H), lambda i: (0, 0)),
           pl.BlockSpec((1, _H), lambda i: (0, 0))]
    xc1 = pl.pallas_call(
        _comp_combine_body,
        grid=(_NC // bc,),
        in_specs=csp,
        out_specs=pl.BlockSpec((bc, _H), lambda i: (i, 0)),
        out_shape=jax.ShapeDtypeStruct((_NC, _H), _F32),
    )(men_sum, men_cnt.reshape(2, _NC, 1), rel_sum,
      rel_cnt.reshape(2, _NC, 1), xc,
      c1_men_Wl, c1_men_bl.reshape(1, _H), c1_men_Wr,
      c1_rel_Wl, c1_rel_bl.reshape(1, _H), c1_rel_Wr,
      ln1c_g.reshape(1, _H), ln1c_b.reshape(1, _H))

    men_sum2, rel_sum2 = _comp_kernel(xn1, xc1, men_src, men_dst,
                                      rel_src, rel_dst)

    fsp = csp + [pl.BlockSpec((_H, 32), lambda i: (0, 0)),
                 pl.BlockSpec((1, 32), lambda i: (0, 0)),
                 pl.BlockSpec((32, 1), lambda i: (0, 0)),
                 pl.BlockSpec((1, 1), lambda i: (0, 0))]
    out = pl.pallas_call(
        _final_body,
        grid=(_NC // bc,),
        in_specs=fsp,
        out_specs=pl.BlockSpec((bc, 1), lambda i: (i, 0)),
        out_shape=jax.ShapeDtypeStruct((_NC, 1), _F32),
    )(men_sum2, men_cnt.reshape(2, _NC, 1), rel_sum2,
      rel_cnt.reshape(2, _NC, 1), xc1,
      c2_men_Wl, c2_men_bl.reshape(1, _H), c2_men_Wr,
      c2_rel_Wl, c2_rel_bl.reshape(1, _H), c2_rel_Wr,
      ln2c_g.reshape(1, _H), ln2c_b.reshape(1, _H),
      cls_W1, cls_b1.reshape(1, 32), cls_W2, cls_b2.reshape(1, 1))
    return out[:, 0]


# speedup vs baseline: 1.1813x; 1.0588x over previous
import functools

import jax
import jax.numpy as jnp
from jax import lax
from jax.experimental import pallas as pl
from jax.experimental.pallas import tpu as pltpu
from jax.experimental.pallas import tpu_sc as plsc

_NN = 50000
_NC = 10000
_H = 64
_HH = 32
_CH = 128
_F32 = jnp.float32

_RT_SIM = 400
_RT_MEN = 200
_RT_REL = 40
_NNP = _NN + 64
_NCP = _NC + 64


def _init_const_bufs(bufs_2d, bufs_1d):
    for ref, val in bufs_2d:
        n_r, n_c = ref.shape

        def body2(i, _, ref=ref, val=val, n_c=n_c):
            for cc in range(n_c // 16):
                ref[i, pl.ds(cc * 16, 16)] = jnp.full((16,), val, _F32)
            return 0

        lax.fori_loop(0, n_r, body2, 0)
    for ref, val in bufs_1d:
        (n,) = ref.shape

        def body1(i, _, ref=ref, val=val):
            ref[pl.ds(i * 16, 16)] = jnp.full((16,), val, _F32)
            return 0

        lax.fori_loop(0, n // 16, body1, 0)


def _span8(n):
    return ((n // 16) + 7) // 8 * 8


def _zero_spmem(t, accum, zero_v):
    n_rows = accum.shape[0]
    span = _span8(n_rows)
    n_chunk = (span + _CH - 1) // _CH

    def body(k, _):
        base = jnp.minimum(t * span + k * _CH, n_rows - _CH)
        pltpu.sync_copy(zero_v, accum.at[pl.ds(base, _CH)])
        return 0

    lax.fori_loop(0, n_chunk, body, 0)


def _seg_pass(tab, src_hbm, dst_hbm, base_row, src_v, dst_v, accum,
              rows, sem_g, sem_s, n_blocks):
    rpb = src_v.shape[0]
    nd = len(rows)
    grp = rpb // nd

    def outer(b, _):
        pltpu.sync_copy(src_hbm.at[pl.ds(base_row + b * rpb, rpb)], src_v)
        pltpu.sync_copy(dst_hbm.at[pl.ds(base_row + b * rpb, rpb)], dst_v)
        for u in range(nd):
            pltpu.async_copy(tab.at[src_v.at[u]], rows[u], sem_g[u])

        def body(q, _):
            for u in range(nd):
                j = q * nd + u
                pltpu.make_async_copy(tab.at[src_v.at[j]], rows[u],
                                      sem_g[u]).wait()
                pltpu.async_copy(rows[u], accum.at[dst_v.at[j]], sem_s[u],
                                 add=True)
            for u in range(nd):
                j = q * nd + u
                pltpu.make_async_copy(rows[u], accum.at[dst_v.at[j]],
                                      sem_s[u]).wait()
                pltpu.async_copy(tab.at[src_v.at[j + nd]], rows[u], sem_g[u])
            return 0

        lax.fori_loop(0, grp - 1, body, 0)
        for u in range(nd):
            j = rpb - nd + u
            pltpu.make_async_copy(tab.at[src_v.at[j]], rows[u],
                                  sem_g[u]).wait()
            pltpu.async_copy(rows[u], accum.at[dst_v.at[j]], sem_s[u],
                             add=True)
        for u in range(nd):
            j = rpb - nd + u
            pltpu.make_async_copy(rows[u], accum.at[dst_v.at[j]],
                                  sem_s[u]).wait()
        return 0

    lax.fori_loop(0, n_blocks, outer, 0)


def _cnt_pass(dst_hbm, base_row, dst_v, cnts, ones_v, sem_c, n_blocks):
    rpb = dst_v.shape[0]

    def outer(b, _):
        pltpu.sync_copy(dst_hbm.at[pl.ds(base_row + b * rpb, rpb)], dst_v)
        for u in range(8):
            pltpu.async_copy(ones_v, cnts.at[dst_v.at[u]], sem_c, add=True)

        def body(j, _):
            pltpu.make_async_copy(ones_v, cnts.at[dst_v.at[j]], sem_c).wait()
            pltpu.async_copy(ones_v, cnts.at[dst_v.at[j + 8]], sem_c,
                             add=True)
            return 0

        lax.fori_loop(0, rpb - 8, body, 0)
        for u in range(8):
            pltpu.make_async_copy(ones_v, cnts.at[dst_v.at[rpb - 8 + u]],
                                  sem_c).wait()
        return 0

    lax.fori_loop(0, n_blocks, outer, 0)


def _copy_out(t, c, accum, out, n_valid):
    span = _span8(n_valid)
    n_chunk = (span + _CH - 1) // _CH

    def body(k, _):
        base = jnp.minimum(t * span + k * _CH, n_valid - _CH)
        pltpu.sync_copy(accum.at[pl.ds(base, _CH)],
                        out.at[c, pl.ds(base, _CH)])
        return 0

    lax.fori_loop(0, n_chunk, body, 0)


def _reduce_copy_out(t, c, accum, bufs, out, n_valid):
    span = _span8(n_valid)
    n_chunk = (span + _CH - 1) // _CH
    n_slab = accum.shape[0] // 4

    def body(k, _):
        base = jnp.minimum(t * span + k * _CH, n_valid - _CH)
        for r in range(4):
            pltpu.sync_copy(accum.at[pl.ds(r * n_slab + base, _CH)], bufs[r])

        def vadd(i, _):
            for h in range(2):
                sl = pl.ds(h * 16, 16)
                bufs[0][i, sl] = (bufs[0][i, sl] + bufs[1][i, sl]
                                  + bufs[2][i, sl] + bufs[3][i, sl])
            return 0

        lax.fori_loop(0, _CH, vadd, 0)
        pltpu.sync_copy(bufs[0], out.at[c, pl.ds(base, _CH)])
        return 0

    lax.fori_loop(0, n_chunk, body, 0)


def _copy_out_flat(t, c, cnts, out, n_valid):
    span = _span8(n_valid)
    n_chunk = (span + _CH - 1) // _CH

    def body(k, _):
        base = jnp.minimum(t * span + k * _CH, n_valid - _CH)
        pltpu.sync_copy(cnts.at[pl.ds(base, _CH)],
                        out.at[pl.ds(c * n_valid + base, _CH)])
        return 0

    lax.fori_loop(0, n_chunk, body, 0)


def _make_cnt_kernel():
    mesh = plsc.VectorSubcoreMesh(core_axis_name="c", subcore_axis_name="s")

    @functools.partial(
        pl.kernel,
        out_type=[jax.ShapeDtypeStruct((2 * _NN,), _F32),
                  jax.ShapeDtypeStruct((2 * _NC,), _F32),
                  jax.ShapeDtypeStruct((2 * _NC,), _F32)],
        mesh=mesh,
        compiler_params=pltpu.CompilerParams(use_tc_tiling_on_sc=False),
        scratch_types=[
            pltpu.VMEM_SHARED((_NNP,), _F32),
            pltpu.VMEM_SHARED((_NCP,), _F32),
            pltpu.VMEM_SHARED((_NCP,), _F32),
            pltpu.VMEM((_RT_REL, _CH), jnp.int32),
            pltpu.VMEM((_CH,), _F32),
            pltpu.VMEM((_CH,), _F32),
            pltpu.SemaphoreType.DMA,
        ],
    )
    def k(sdst_hbm, mdst_hbm, rdst_hbm, cs_out, cm_out, cr_out,
          cs, cm, cr, dst_v, zero1_v, ones_v, sem_c):
        c = lax.axis_index("c")
        s = lax.axis_index("s")
        w = s * 2 + c
        _init_const_bufs([], [(zero1_v, 0.0), (ones_v, 1.0)])
        _zero_spmem(s, cs, zero1_v)
        _zero_spmem(s, cm, zero1_v)
        _zero_spmem(s, cr, zero1_v)
        plsc.subcore_barrier()
        _cnt_pass(sdst_hbm, w * _RT_MEN, dst_v, cs, ones_v, sem_c, 5)
        _cnt_pass(mdst_hbm, w * _RT_MEN, dst_v, cm, ones_v, sem_c, 5)
        _cnt_pass(rdst_hbm, w * _RT_REL, dst_v, cr, ones_v, sem_c, 1)
        plsc.subcore_barrier()
        _copy_out_flat(s, c, cs, cs_out, _NN)
        _copy_out_flat(s, c, cm, cm_out, _NC)
        _copy_out_flat(s, c, cr, cr_out, _NC)

    return k


def _make_sim_kernel():
    mesh = plsc.VectorSubcoreMesh(core_axis_name="c", subcore_axis_name="s")

    @functools.partial(
        pl.kernel,
        out_type=jax.ShapeDtypeStruct((2, _NN, _HH), _F32),
        mesh=mesh,
        compiler_params=pltpu.CompilerParams(use_tc_tiling_on_sc=False),
        scratch_types=[
            pltpu.VMEM_SHARED((_NNP, _HH), _F32),
            pltpu.VMEM((40, _CH), jnp.int32),
            pltpu.VMEM((40, _CH), jnp.int32),
            pltpu.VMEM((_CH, _HH), _F32),
            pltpu.VMEM((_CH, _HH), _F32),
            pltpu.VMEM((_CH, _HH), _F32),
            pltpu.VMEM((_CH, _HH), _F32),
        ] + [pltpu.SemaphoreType.DMA] * 8,
    )
    def k(lo_hbm, hi_hbm, src_hbm, dst_hbm, sum_out,
          accum, src_v, dst_v, r0, r1, r2, r3,
          sg0, sg1, sg2, sg3, ss0, ss1, ss2, ss3):
        rows = [r0, r1, r2, r3]
        sem_g = [sg0, sg1, sg2, sg3]
        sem_s = [ss0, ss1, ss2, ss3]
        c = lax.axis_index("c")
        s = lax.axis_index("s")
        _init_const_bufs([(r0, 0.0)], [])
        _zero_spmem(s, accum, r0)
        plsc.subcore_barrier()

        @pl.when(c == 0)
        def _():
            _seg_pass(lo_hbm, src_hbm, dst_hbm, s * _RT_SIM, src_v, dst_v,
                      accum, rows, sem_g, sem_s, _RT_SIM // 40)

        @pl.when(c == 1)
        def _():
            _seg_pass(hi_hbm, src_hbm, dst_hbm, s * _RT_SIM, src_v, dst_v,
                      accum, rows, sem_g, sem_s, _RT_SIM // 40)

        plsc.subcore_barrier()
        _copy_out(s, c, accum, sum_out, _NN)

    return k


def _make_single_rel_kernel(n_src_rows_unused, rt, ring):
    mesh = plsc.VectorSubcoreMesh(core_axis_name="c", subcore_axis_name="s")

    @functools.partial(
        pl.kernel,
        out_type=jax.ShapeDtypeStruct((2, _NC, _H), _F32),
        mesh=mesh,
        compiler_params=pltpu.CompilerParams(use_tc_tiling_on_sc=False),
        scratch_types=[
            pltpu.VMEM_SHARED((_NCP, _H), _F32),
            pltpu.VMEM((_RT_REL, _CH), jnp.int32),
            pltpu.VMEM((_RT_REL, _CH), jnp.int32),
        ] + [pltpu.VMEM((_CH, _H), _F32)] * ring
          + [pltpu.SemaphoreType.DMA] * (2 * ring),
    )
    def k(tab_hbm, src_hbm, dst_hbm, sum_out, acc, src_v, dst_v, *rest):
        rows = list(rest[:ring])
        sem_g = list(rest[ring:2 * ring])
        sem_s = list(rest[2 * ring:])
        c = lax.axis_index("c")
        s = lax.axis_index("s")
        w = s * 2 + c
        _init_const_bufs([(rows[0], 0.0)], [])
        _zero_spmem(s, acc, rows[0])
        plsc.subcore_barrier()
        _seg_pass(tab_hbm, src_hbm, dst_hbm, w * rt, src_v, dst_v,
                  acc, rows, sem_g, sem_s, rt // _RT_REL)
        plsc.subcore_barrier()
        _copy_out(s, c, acc, sum_out, _NC)

    return k


@functools.cache
def _get_sc_kernels():
    return (_make_cnt_kernel(), _make_sim_kernel(),
            _make_single_rel_kernel(0, _RT_MEN, 8),
            _make_single_rel_kernel(0, _RT_REL, 4))


def _pad_edges(ei, n_rows_total, n_dst):
    e = ei.shape[1]
    npad = n_rows_total * _CH - e
    pad = jnp.arange(npad, dtype=jnp.int32)
    src = jnp.concatenate([ei[0], pad % 1024]).reshape(n_rows_total, _CH)
    dst = jnp.concatenate([ei[1], n_dst + pad % 64]).reshape(n_rows_total,
                                                             _CH)
    return src, dst


def _ln(x, g, b):
    mu = jnp.mean(x, axis=-1, keepdims=True)
    v = jnp.mean((x - mu) ** 2, axis=-1, keepdims=True)
    return (x - mu) * jax.lax.rsqrt(v + 1e-5) * g + b


def _nproj_body(x_ref, w_ref, b_ref, xn_ref, lo_ref, hi_ref):
    y = jnp.maximum(
        jnp.dot(x_ref[...], w_ref[...], preferred_element_type=_F32)
        + b_ref[...], 0.0)
    xn_ref[...] = y
    lo_ref[...] = y[:, :_HH]
    hi_ref[...] = y[:, _HH:]


def _cproj_body(x_ref, w_ref, b_ref, xc_ref):
    xc_ref[...] = jnp.maximum(
        jnp.dot(x_ref[...], w_ref[...], preferred_element_type=_F32)
        + b_ref[...], 0.0)


def _news_combine_body(lo_ref, hi_ref, c0_ref, c1_ref, xn_ref,
                       wl_ref, bl_ref, wr_ref, g_ref, b_ref, out_ref):
    cnt = jnp.maximum(c0_ref[...] + c1_ref[...], 1.0)
    mean = jnp.concatenate([lo_ref[...], hi_ref[...]], axis=1) / cnt
    n1 = (jnp.dot(mean, wl_ref[...], preferred_element_type=_F32)
          + bl_ref[...]
          + jnp.dot(xn_ref[...], wr_ref[...], preferred_element_type=_F32))
    out_ref[...] = _ln(jnp.maximum(n1, 0.0), g_ref[...], b_ref[...])


def _comp_means(sm_ref, cm_ref, sr_ref, cr_ref):
    sm = sm_ref[...]
    sr = sr_ref[...]
    mm = (sm[0] + sm[1]) / jnp.maximum(cm_ref[...][0] + cm_ref[...][1], 1.0)
    mr = (sr[0] + sr[1]) / jnp.maximum(cr_ref[...][0] + cr_ref[...][1], 1.0)
    return mm, mr


def _comp_combine_body(sm_ref, cm_ref, sr_ref, cr_ref, xc_ref,
                       wlm_ref, blm_ref, wrm_ref,
                       wlr_ref, blr_ref, wrr_ref,
                       g_ref, b_ref, out_ref):
    mm, mr = _comp_means(sm_ref, cm_ref, sr_ref, cr_ref)
    xc = xc_ref[...]
    cc = 0.5 * (
        jnp.dot(mm, wlm_ref[...], preferred_element_type=_F32) + blm_ref[...]
        + jnp.dot(xc, wrm_ref[...], preferred_element_type=_F32)
        + jnp.dot(mr, wlr_ref[...], preferred_element_type=_F32)
        + blr_ref[...]
        + jnp.dot(xc, wrr_ref[...], preferred_element_type=_F32))
    out_ref[...] = _ln(jnp.maximum(cc, 0.0), g_ref[...], b_ref[...])


def _final_body(sm_ref, cm_ref, sr_ref, cr_ref, xc_ref,
                wlm_ref, blm_ref, wrm_ref, wlr_ref, blr_ref, wrr_ref,
                g_ref, b_ref, w1_ref, b1_ref, w2_ref, b2_ref, out_ref):
    mm, mr = _comp_means(sm_ref, cm_ref, sr_ref, cr_ref)
    xc = xc_ref[...]
    cc = 0.5 * (
        jnp.dot(mm, wlm_ref[...], preferred_element_type=_F32) + blm_ref[...]
        + jnp.dot(xc, wrm_ref[...], preferred_element_type=_F32)
        + jnp.dot(mr, wlr_ref[...], preferred_element_type=_F32)
        + blr_ref[...]
        + jnp.dot(xc, wrr_ref[...], preferred_element_type=_F32))
    x2 = _ln(jnp.maximum(cc, 0.0), g_ref[...], b_ref[...])
    h = jnp.maximum(
        jnp.dot(x2, w1_ref[...], preferred_element_type=_F32) + b1_ref[...],
        0.0)
    out_ref[...] = (jnp.dot(h, w2_ref[...], preferred_element_type=_F32)
                    + b2_ref[...])


def kernel(news_x, company_x, edge_sim, edge_men, edge_rel, news_proj_W, news_proj_b, company_proj_W, company_proj_b, c1_sim_Wl, c1_sim_bl, c1_sim_Wr, c1_men_Wl, c1_men_bl, c1_men_Wr, c1_rel_Wl, c1_rel_bl, c1_rel_Wr, c2_sim_Wl, c2_sim_bl, c2_sim_Wr, c2_men_Wl, c2_men_bl, c2_men_Wr, c2_rel_Wl, c2_rel_bl, c2_rel_Wr, ln1n_g, ln1n_b, ln1c_g, ln1c_b, ln2c_g, ln2c_b, cls_W1, cls_b1, cls_W2, cls_b2):
    _cnt_kernel, _sim_kernel, _men_kernel, _rel_kernel = _get_sc_kernels()
    sim_src, sim_dst = _pad_edges(edge_sim, 16 * _RT_SIM, _NN)
    men_src, men_dst = _pad_edges(edge_men, 32 * _RT_MEN, _NC)
    rel_src, rel_dst = _pad_edges(edge_rel, 32 * _RT_REL, _NC)
    sim_cnt, men_cnt, rel_cnt = _cnt_kernel(sim_dst, men_dst, rel_dst)

    news_x, company_x, _, _, _ = lax.optimization_barrier(
        (news_x, company_x, sim_dst, men_dst, rel_dst))

    bm = 5000
    xn, xn_lo, xn_hi = pl.pallas_call(
        _nproj_body,
        grid=(_NN // bm,),
        in_specs=[pl.BlockSpec((bm, 385), lambda i: (i, 0)),
                  pl.BlockSpec((385, _H), lambda i: (0, 0)),
                  pl.BlockSpec((1, _H), lambda i: (0, 0))],
        out_specs=[pl.BlockSpec((bm, _H), lambda i: (i, 0)),
                   pl.BlockSpec((bm, _HH), lambda i: (i, 0)),
                   pl.BlockSpec((bm, _HH), lambda i: (i, 0))],
        out_shape=[jax.ShapeDtypeStruct((_NN, _H), _F32),
                   jax.ShapeDtypeStruct((_NN, _HH), _F32),
                   jax.ShapeDtypeStruct((_NN, _HH), _F32)],
    )(news_x, news_proj_W, news_proj_b.reshape(1, _H))
    xc = pl.pallas_call(
        _cproj_body,
        out_shape=jax.ShapeDtypeStruct((_NC, _H), _F32),
    )(company_x, company_proj_W, company_proj_b.reshape(1, _H))

    sim_sum = _sim_kernel(xn_lo, xn_hi, sim_src, sim_dst)
    men_sum = _men_kernel(xn, men_src, men_dst)
    rel_sum = _rel_kernel(xc, rel_src, rel_dst)

    xn1 = pl.pallas_call(
        _news_combine_body,
        grid=(_NN // bm,),
        in_specs=[pl.BlockSpec((bm, _HH), lambda i: (i, 0)),
                  pl.BlockSpec((bm, _HH), lambda i: (i, 0)),
                  pl.BlockSpec((bm, 1), lambda i: (i, 0)),
                  pl.BlockSpec((bm, 1), lambda i: (i, 0)),
                  pl.BlockSpec((bm, _H), lambda i: (i, 0)),
                  pl.BlockSpec((_H, _H), lambda i: (0, 0)),
                  pl.BlockSpec((1, _H), lambda i: (0, 0)),
                  pl.BlockSpec((_H, _H), lambda i: (0, 0)),
                  pl.BlockSpec((1, _H), lambda i: (0, 0)),
                  pl.BlockSpec((1, _H), lambda i: (0, 0))],
        out_specs=pl.BlockSpec((bm, _H), lambda i: (i, 0)),
        out_shape=jax.ShapeDtypeStruct((_NN, _H), _F32),
    )(sim_sum[0], sim_sum[1],
      sim_cnt[:_NN].reshape(_NN, 1), sim_cnt[_NN:].reshape(_NN, 1), xn,
      c1_sim_Wl, c1_sim_bl.reshape(1, _H), c1_sim_Wr,
      ln1n_g.reshape(1, _H), ln1n_b.reshape(1, _H))
    bc = 2000
    csp = [pl.BlockSpec((2, bc, _H), lambda i: (0, i, 0)),
           pl.BlockSpec((2, bc, 1), lambda i: (0, i, 0)),
           pl.BlockSpec((2, bc, _H), lambda i: (0, i, 0)),
           pl.BlockSpec((2, bc, 1), lambda i: (0, i, 0)),
           pl.BlockSpec((bc, _H), lambda i: (i, 0)),
           pl.BlockSpec((_H, _H), lambda i: (0, 0)),
           pl.BlockSpec((1, _H), lambda i: (0, 0)),
           pl.BlockSpec((_H, _H), lambda i: (0, 0)),
           pl.BlockSpec((_H, _H), lambda i: (0, 0)),
           pl.BlockSpec((1, _H), lambda i: (0, 0)),
           pl.BlockSpec((_H, _H), lambda i: (0, 0)),
           pl.BlockSpec((1, _H), lambda i: (0, 0)),
           pl.BlockSpec((1, _H), lambda i: (0, 0))]
    xc1 = pl.pallas_call(
        _comp_combine_body,
        grid=(_NC // bc,),
        in_specs=csp,
        out_specs=pl.BlockSpec((bc, _H), lambda i: (i, 0)),
        out_shape=jax.ShapeDtypeStruct((_NC, _H), _F32),
    )(men_sum, men_cnt.reshape(2, _NC, 1), rel_sum,
      rel_cnt.reshape(2, _NC, 1), xc,
      c1_men_Wl, c1_men_bl.reshape(1, _H), c1_men_Wr,
      c1_rel_Wl, c1_rel_bl.reshape(1, _H), c1_rel_Wr,
      ln1c_g.reshape(1, _H), ln1c_b.reshape(1, _H))

    men_sum2 = _men_kernel(xn1, men_src, men_dst)
    rel_sum2 = _rel_kernel(xc1, rel_src, rel_dst)

    fsp = csp + [pl.BlockSpec((_H, 32), lambda i: (0, 0)),
                 pl.BlockSpec((1, 32), lambda i: (0, 0)),
                 pl.BlockSpec((32, 1), lambda i: (0, 0)),
                 pl.BlockSpec((1, 1), lambda i: (0, 0))]
    out = pl.pallas_call(
        _final_body,
        grid=(_NC // bc,),
        in_specs=fsp,
        out_specs=pl.BlockSpec((bc, 1), lambda i: (i, 0)),
        out_shape=jax.ShapeDtypeStruct((_NC, 1), _F32),
    )(men_sum2, men_cnt.reshape(2, _NC, 1), rel_sum2,
      rel_cnt.reshape(2, _NC, 1), xc1,
      c2_men_Wl, c2_men_bl.reshape(1, _H), c2_men_Wr,
      c2_rel_Wl, c2_rel_bl.reshape(1, _H), c2_rel_Wr,
      ln2c_g.reshape(1, _H), ln2c_b.reshape(1, _H),
      cls_W1, cls_b1.reshape(1, 32), cls_W2, cls_b2.reshape(1, 1))
    return out[:, 0]
